# baseline (speedup 1.0000x reference)
import functools

import jax
import jax.numpy as jnp
from jax import lax
from jax.experimental import pallas as pl
from jax.experimental.pallas import tpu as pltpu
from jax.experimental.pallas import tpu_sc as plsc

N = 4096
D = 1024
C = 1000
E = 8

BP = 128
BPLOG = 7
NP = N + E * BP
NBLK = NP // BP
NBLKP = 48
CP = 1024
TPS = 256
NLANE = 16


def _sc_mesh():
    return plsc.VectorSubcoreMesh(core_axis_name="c", subcore_axis_name="s")


def _route_body(sid_hbm, x_hbm, xs_hbm, dst_hbm, blk_hbm,
                sid_v, rank_v, dst_v, cnt_v, allc_v, start_v, off_v, blk_v,
                xbuf_a, xbuf_b, sh_cnt, sem_a, sem_b):
    c = lax.axis_index("c")
    s = lax.axis_index("s")
    base = s * TPS
    pltpu.sync_copy(sid_hbm.at[pl.ds(base, TPS)], sid_v)

    lane = lax.iota(jnp.int32, NLANE)
    counts = [jnp.zeros((NLANE,), jnp.int32) for _ in range(E)]
    for i in range(TPS // NLANE):
        v = sid_v[pl.ds(i * NLANE, NLANE)]
        rank = jnp.zeros((NLANE,), jnp.int32)
        for e in range(E):
            m = v == e
            cs = plsc.cumsum(m.astype(jnp.int32))
            rank = jnp.where(m, counts[e] + cs - 1, rank)
            counts[e] = counts[e] + plsc.all_reduce_population_count(m)
        rank_v[pl.ds(i * NLANE, NLANE)] = rank

    cvec = jnp.zeros((NLANE,), jnp.int32)
    for e in range(E):
        cvec = jnp.where(lane == e, counts[e], cvec)
    cnt_v[...] = cvec
    pltpu.sync_copy(cnt_v, sh_cnt.at[pl.ds(s * NLANE, NLANE)])
    plsc.subcore_barrier()
    pltpu.sync_copy(sh_cnt, allc_v)

    total = jnp.zeros((NLANE,), jnp.int32)
    prior = jnp.zeros((NLANE,), jnp.int32)
    for t in range(16):
        row = allc_v[pl.ds(t * NLANE, NLANE)]
        total = total + row
        tm = jnp.full((NLANE,), t, jnp.int32) < s
        prior = prior + jnp.where(tm, row, jnp.zeros((NLANE,), jnp.int32))

    pe = ((total + (BP - 1)) >> BPLOG) << BPLOG
    off = plsc.cumsum(pe) - pe
    start_v[...] = off + prior
    off_v[...] = off

    for i in range(TPS // NLANE):
        v = sid_v[pl.ds(i * NLANE, NLANE)]
        g = plsc.load_gather(start_v, [v])
        dst_v[pl.ds(i * NLANE, NLANE)] = g + rank_v[pl.ds(i * NLANE, NLANE)]

    for cb in (0, 16, 32):
        kvec = lax.iota(jnp.int32, NLANE) + cb
        sp = kvec * BP
        gid = jnp.zeros((NLANE,), jnp.int32)
        for e in range(1, E):
            be = plsc.load_gather(off_v, [jnp.full((NLANE,), e, jnp.int32)])
            gid = gid + (sp >= be).astype(jnp.int32)
        blk_v[pl.ds(cb, NLANE)] = gid

    @pl.when(jnp.logical_and(c == 0, s == 0))
    def _():
        pltpu.sync_copy(blk_v, blk_hbm)

    def do_half(lo):
        pltpu.sync_copy(dst_v.at[pl.ds(lo, 128)],
                        dst_hbm.at[pl.ds(base + lo, 128)])
        bufs = (xbuf_a, xbuf_b)
        sems = (sem_a, sem_b)
        loads = [None, None]
        loads[0] = pltpu.async_copy(
            x_hbm.at[pl.ds(base + lo, NLANE)], bufs[0], sems[0])
        for j in range(8):
            p = j % 2
            if j < 7:
                loads[1 - p] = pltpu.async_copy(
                    x_hbm.at[pl.ds(base + lo + (j + 1) * NLANE, NLANE)],
                    bufs[1 - p], sems[1 - p])
            loads[p].wait()
            idx = dst_v[pl.ds(lo + j * NLANE, NLANE)]
            pltpu.async_copy(bufs[p], xs_hbm.at[idx], sems[p]).wait()

    @pl.when(c == 0)
    def _():
        do_half(0)

    @pl.when(c == 1)
    def _():
        do_half(128)


def _route_and_scatter(sid, x):
    f = pl.kernel(
        _route_body,
        compiler_params=pltpu.CompilerParams(needs_layout_passes=False),
        out_type=(
            jax.ShapeDtypeStruct((NP, D), jnp.float32),
            jax.ShapeDtypeStruct((N,), jnp.int32),
            jax.ShapeDtypeStruct((NBLKP,), jnp.int32),
        ),
        mesh=_sc_mesh(),
        scratch_types=[
            pltpu.VMEM((TPS,), jnp.int32),
            pltpu.VMEM((TPS,), jnp.int32),
            pltpu.VMEM((TPS,), jnp.int32),
            pltpu.VMEM((NLANE,), jnp.int32),
            pltpu.VMEM((256,), jnp.int32),
            pltpu.VMEM((NLANE,), jnp.int32),
            pltpu.VMEM((NLANE,), jnp.int32),
            pltpu.VMEM((NBLKP,), jnp.int32),
            pltpu.VMEM((NLANE, D), jnp.float32),
            pltpu.VMEM((NLANE, D), jnp.float32),
            pltpu.VMEM_SHARED((256,), jnp.int32),
            pltpu.SemaphoreType.DMA,
            pltpu.SemaphoreType.DMA,
        ],
    )
    return f(sid, x)


def _mm_body(blk_ref, x_ref, w_ref, b_ref, y_ref):
    del blk_ref
    logits = jax.lax.dot_general(
        x_ref[...].astype(jnp.bfloat16), w_ref[0].astype(jnp.bfloat16),
        (((1,), (1,)), ((), ())),
        preferred_element_type=jnp.float32,
    ) + b_ref[0]
    y_ref[...] = jnp.concatenate(
        [logits, jnp.zeros((BP, CP - C), jnp.float32)], axis=1)


def _grouped_matmul(blk, xs, W, b3):
    grid_spec = pltpu.PrefetchScalarGridSpec(
        num_scalar_prefetch=1,
        grid=(NBLK,),
        in_specs=[
            pl.BlockSpec((BP, D), lambda k, g: (k, 0)),
            pl.BlockSpec((1, C, D), lambda k, g: (g[k], 0, 0)),
            pl.BlockSpec((1, 1, C), lambda k, g: (g[k], 0, 0)),
        ],
        out_specs=pl.BlockSpec((BP, CP), lambda k, g: (k, 0)),
    )
    return pl.pallas_call(
        _mm_body,
        grid_spec=grid_spec,
        out_shape=jax.ShapeDtypeStruct((NP, CP), jnp.float32),
        compiler_params=pltpu.CompilerParams(
            dimension_semantics=("arbitrary",),
        ),
    )(blk, xs, W, b3)


def _gather_body(y_hbm, dst_hbm, out_hbm, dst_v, rows_a, rows_b,
                 sem_a, sem_b):
    c = lax.axis_index("c")
    s = lax.axis_index("s")
    tok0 = s * TPS + c * 128
    pltpu.sync_copy(dst_hbm.at[pl.ds(tok0, 128)], dst_v)
    bufs = (rows_a, rows_b)
    sems = (sem_a, sem_b)
    loads = [None, None]
    idx0 = dst_v[pl.ds(0, NLANE)]
    loads[0] = pltpu.async_copy(y_hbm.at[idx0], bufs[0], sems[0])
    for j in range(8):
        p = j % 2
        if j < 7:
            idx = dst_v[pl.ds((j + 1) * NLANE, NLANE)]
            loads[1 - p] = pltpu.async_copy(y_hbm.at[idx], bufs[1 - p], sems[1 - p])
        loads[p].wait()
        pltpu.sync_copy(bufs[p], out_hbm.at[pl.ds(tok0 + j * NLANE, NLANE)])


def _gather_back(y, dst):
    f = pl.kernel(
        _gather_body,
        compiler_params=pltpu.CompilerParams(needs_layout_passes=False),
        out_type=jax.ShapeDtypeStruct((N, CP), jnp.float32),
        mesh=_sc_mesh(),
        scratch_types=[
            pltpu.VMEM((128,), jnp.int32),
            pltpu.VMEM((NLANE, CP), jnp.float32),
            pltpu.VMEM((NLANE, CP), jnp.float32),
            pltpu.SemaphoreType.DMA,
            pltpu.SemaphoreType.DMA,
        ],
    )
    return f(y, dst)


def _trim_body(full_ref, out_ref):
    out_ref[...] = full_ref[:, :C]


def _trim(full):
    bn = 512
    return pl.pallas_call(
        _trim_body,
        grid=(N // bn,),
        in_specs=[pl.BlockSpec((bn, CP), lambda i: (i, 0))],
        out_specs=pl.BlockSpec((bn, C), lambda i: (i, 0)),
        out_shape=jax.ShapeDtypeStruct((N, C), jnp.float32),
    )(full)


def kernel(x, system_id, W, b):
    sid = system_id.astype(jnp.int32)
    b3 = b.reshape(E, 1, C)
    xs, dst, blk = _route_and_scatter(sid, x)
    y = _grouped_matmul(blk, xs, W, b3)
    return _trim(_gather_back(y, dst))


# speedup vs baseline: 1.0786x; 1.0786x over previous
import functools

import jax
import jax.numpy as jnp
from jax import lax
from jax.experimental import pallas as pl
from jax.experimental.pallas import tpu as pltpu
from jax.experimental.pallas import tpu_sc as plsc

N = 4096
D = 1024
C = 1000
E = 8

BP = 256
BPLOG = 8
NP = N + E * BP
NBLK = NP // BP
NBLKP = 32
CP = 1024
TPS = 256
NLANE = 16


def _sc_mesh():
    return plsc.VectorSubcoreMesh(core_axis_name="c", subcore_axis_name="s")


def _route_body(sid_hbm, x_hbm, xs_hbm, dst_hbm, blk_hbm,
                sid_v, rank_v, dst_v, cnt_v, allc_v, start_v, off_v, blk_v,
                xbuf_a, xbuf_b, sh_cnt, sem_a, sem_b):
    c = lax.axis_index("c")
    s = lax.axis_index("s")
    base = s * TPS
    pltpu.sync_copy(sid_hbm.at[pl.ds(base, TPS)], sid_v)

    lane = lax.iota(jnp.int32, NLANE)
    counts = [jnp.zeros((NLANE,), jnp.int32) for _ in range(E)]
    for i in range(TPS // NLANE):
        v = sid_v[pl.ds(i * NLANE, NLANE)]
        rank = jnp.zeros((NLANE,), jnp.int32)
        for e in range(E):
            m = v == e
            cs = plsc.cumsum(m.astype(jnp.int32))
            rank = jnp.where(m, counts[e] + cs - 1, rank)
            counts[e] = counts[e] + plsc.all_reduce_population_count(m)
        rank_v[pl.ds(i * NLANE, NLANE)] = rank

    cvec = jnp.zeros((NLANE,), jnp.int32)
    for e in range(E):
        cvec = jnp.where(lane == e, counts[e], cvec)
    cnt_v[...] = cvec
    pltpu.sync_copy(cnt_v, sh_cnt.at[pl.ds(s * NLANE, NLANE)])
    plsc.subcore_barrier()
    pltpu.sync_copy(sh_cnt, allc_v)

    total = jnp.zeros((NLANE,), jnp.int32)
    prior = jnp.zeros((NLANE,), jnp.int32)
    for t in range(16):
        row = allc_v[pl.ds(t * NLANE, NLANE)]
        total = total + row
        tm = jnp.full((NLANE,), t, jnp.int32) < s
        prior = prior + jnp.where(tm, row, jnp.zeros((NLANE,), jnp.int32))

    pe = ((total + (BP - 1)) >> BPLOG) << BPLOG
    off = plsc.cumsum(pe) - pe
    start_v[...] = off + prior
    off_v[...] = off

    for i in range(TPS // NLANE):
        v = sid_v[pl.ds(i * NLANE, NLANE)]
        g = plsc.load_gather(start_v, [v])
        dst_v[pl.ds(i * NLANE, NLANE)] = g + rank_v[pl.ds(i * NLANE, NLANE)]

    for cb in (0, 16):
        kvec = lax.iota(jnp.int32, NLANE) + cb
        sp = kvec * BP
        gid = jnp.zeros((NLANE,), jnp.int32)
        for e in range(1, E):
            be = plsc.load_gather(off_v, [jnp.full((NLANE,), e, jnp.int32)])
            gid = gid + (sp >= be).astype(jnp.int32)
        blk_v[pl.ds(cb, NLANE)] = gid

    @pl.when(jnp.logical_and(c == 0, s == 0))
    def _():
        pltpu.sync_copy(blk_v, blk_hbm)

    def do_half(lo):
        pltpu.sync_copy(dst_v.at[pl.ds(lo, 128)],
                        dst_hbm.at[pl.ds(base + lo, 128)])
        bufs = (xbuf_a, xbuf_b)
        sems = (sem_a, sem_b)
        loads = [None, None]
        loads[0] = pltpu.async_copy(
            x_hbm.at[pl.ds(base + lo, NLANE)], bufs[0], sems[0])
        for j in range(8):
            p = j % 2
            if j < 7:
                loads[1 - p] = pltpu.async_copy(
                    x_hbm.at[pl.ds(base + lo + (j + 1) * NLANE, NLANE)],
                    bufs[1 - p], sems[1 - p])
            loads[p].wait()
            idx = dst_v[pl.ds(lo + j * NLANE, NLANE)]
            pltpu.async_copy(bufs[p], xs_hbm.at[idx], sems[p]).wait()

    @pl.when(c == 0)
    def _():
        do_half(0)

    @pl.when(c == 1)
    def _():
        do_half(128)


def _route_and_scatter(sid, x):
    f = pl.kernel(
        _route_body,
        compiler_params=pltpu.CompilerParams(needs_layout_passes=False),
        out_type=(
            jax.ShapeDtypeStruct((NP, D), jnp.float32),
            jax.ShapeDtypeStruct((N,), jnp.int32),
            jax.ShapeDtypeStruct((NBLKP,), jnp.int32),
        ),
        mesh=_sc_mesh(),
        scratch_types=[
            pltpu.VMEM((TPS,), jnp.int32),
            pltpu.VMEM((TPS,), jnp.int32),
            pltpu.VMEM((TPS,), jnp.int32),
            pltpu.VMEM((NLANE,), jnp.int32),
            pltpu.VMEM((256,), jnp.int32),
            pltpu.VMEM((NLANE,), jnp.int32),
            pltpu.VMEM((NLANE,), jnp.int32),
            pltpu.VMEM((NBLKP,), jnp.int32),
            pltpu.VMEM((NLANE, D), jnp.float32),
            pltpu.VMEM((NLANE, D), jnp.float32),
            pltpu.VMEM_SHARED((256,), jnp.int32),
            pltpu.SemaphoreType.DMA,
            pltpu.SemaphoreType.DMA,
        ],
    )
    return f(sid, x)


def _mm_body(blk_ref, x_ref, w_ref, b_ref, y_ref):
    del blk_ref
    logits = jax.lax.dot_general(
        x_ref[...].astype(jnp.bfloat16), w_ref[0],
        (((1,), (1,)), ((), ())),
        preferred_element_type=jnp.float32,
    ) + b_ref[0]
    y_ref[...] = jnp.concatenate(
        [logits, jnp.zeros((BP, CP - C), jnp.float32)], axis=1)


def _grouped_matmul(blk, xs, W, b3):
    grid_spec = pltpu.PrefetchScalarGridSpec(
        num_scalar_prefetch=1,
        grid=(NBLK,),
        in_specs=[
            pl.BlockSpec((BP, D), lambda k, g: (k, 0)),
            pl.BlockSpec((1, C, D), lambda k, g: (g[k], 0, 0)),
            pl.BlockSpec((1, 1, C), lambda k, g: (g[k], 0, 0)),
        ],
        out_specs=pl.BlockSpec((BP, CP), lambda k, g: (k, 0)),
    )
    return pl.pallas_call(
        _mm_body,
        grid_spec=grid_spec,
        out_shape=jax.ShapeDtypeStruct((NP, CP), jnp.float32),
        compiler_params=pltpu.CompilerParams(
            dimension_semantics=("arbitrary",),
        ),
    )(blk, xs, W, b3)


def _gather_body(y_hbm, dst_hbm, out_hbm, dst_v, rows_a, rows_b,
                 sem_a, sem_b):
    c = lax.axis_index("c")
    s = lax.axis_index("s")
    tok0 = s * TPS + c * 128
    pltpu.sync_copy(dst_hbm.at[pl.ds(tok0, 128)], dst_v)
    bufs = (rows_a, rows_b)
    sems = (sem_a, sem_b)
    loads = [None, None]
    idx0 = dst_v[pl.ds(0, NLANE)]
    loads[0] = pltpu.async_copy(y_hbm.at[idx0], bufs[0], sems[0])
    for j in range(8):
        p = j % 2
        if j < 7:
            idx = dst_v[pl.ds((j + 1) * NLANE, NLANE)]
            loads[1 - p] = pltpu.async_copy(y_hbm.at[idx], bufs[1 - p], sems[1 - p])
        loads[p].wait()
        pltpu.sync_copy(bufs[p], out_hbm.at[pl.ds(tok0 + j * NLANE, NLANE)])


def _gather_back(y, dst):
    f = pl.kernel(
        _gather_body,
        compiler_params=pltpu.CompilerParams(needs_layout_passes=False),
        out_type=jax.ShapeDtypeStruct((N, CP), jnp.float32),
        mesh=_sc_mesh(),
        scratch_types=[
            pltpu.VMEM((128,), jnp.int32),
            pltpu.VMEM((NLANE, CP), jnp.float32),
            pltpu.VMEM((NLANE, CP), jnp.float32),
            pltpu.SemaphoreType.DMA,
            pltpu.SemaphoreType.DMA,
        ],
    )
    return f(y, dst)


def _trim_body(full_ref, out_ref):
    out_ref[...] = full_ref[:, :C]


def _trim(full):
    bn = 512
    return pl.pallas_call(
        _trim_body,
        grid=(N // bn,),
        in_specs=[pl.BlockSpec((bn, CP), lambda i: (i, 0))],
        out_specs=pl.BlockSpec((bn, C), lambda i: (i, 0)),
        out_shape=jax.ShapeDtypeStruct((N, C), jnp.float32),
    )(full)


def kernel(x, system_id, W, b):
    sid = system_id.astype(jnp.int32)
    b3 = b.reshape(E, 1, C)
    W16 = W.astype(jnp.bfloat16)
    xs, dst, blk = _route_and_scatter(sid, x)
    y = _grouped_matmul(blk, xs, W16, b3)
    return _trim(_gather_back(y, dst))


# speedup vs baseline: 1.1274x; 1.0452x over previous
import functools

import jax
import jax.numpy as jnp
from jax import lax
from jax.experimental import pallas as pl
from jax.experimental.pallas import tpu as pltpu
from jax.experimental.pallas import tpu_sc as plsc

N = 4096
D = 1024
C = 1000
E = 8

BP = 256
BPLOG = 8
NP = N + E * BP
NBLK = NP // BP
NBLKP = 32
CP = 1024
TPS = 256
NLANE = 16


def _sc_mesh():
    return plsc.VectorSubcoreMesh(core_axis_name="c", subcore_axis_name="s")


def _route_body(sid_hbm, x_hbm, xs_hbm, dst_hbm, blk_hbm,
                sid_v, rank_v, dst_v, cnt_v, allc_v, start_v, off_v, end_v,
                blk_v, xbuf_a, xbuf_b, sh_cnt, sem_a, sem_b):
    c = lax.axis_index("c")
    s = lax.axis_index("s")
    base = s * TPS
    pltpu.sync_copy(sid_hbm.at[pl.ds(base, TPS)], sid_v)

    lane = lax.iota(jnp.int32, NLANE)
    counts = [jnp.zeros((NLANE,), jnp.int32) for _ in range(E)]
    for i in range(TPS // NLANE):
        v = sid_v[pl.ds(i * NLANE, NLANE)]
        rank = jnp.zeros((NLANE,), jnp.int32)
        for e in range(E):
            m = v == e
            cs = plsc.cumsum(m.astype(jnp.int32))
            rank = jnp.where(m, counts[e] + cs - 1, rank)
            counts[e] = counts[e] + plsc.all_reduce_population_count(m)
        rank_v[pl.ds(i * NLANE, NLANE)] = rank

    cvec = jnp.zeros((NLANE,), jnp.int32)
    for e in range(E):
        cvec = jnp.where(lane == e, counts[e], cvec)
    cnt_v[...] = cvec
    pltpu.sync_copy(cnt_v, sh_cnt.at[pl.ds(s * NLANE, NLANE)])
    plsc.subcore_barrier()
    pltpu.sync_copy(sh_cnt, allc_v)

    total = jnp.zeros((NLANE,), jnp.int32)
    prior = jnp.zeros((NLANE,), jnp.int32)
    for t in range(16):
        row = allc_v[pl.ds(t * NLANE, NLANE)]
        total = total + row
        tm = jnp.full((NLANE,), t, jnp.int32) < s
        prior = prior + jnp.where(tm, row, jnp.zeros((NLANE,), jnp.int32))

    pe = ((total + (BP - 1)) >> BPLOG) << BPLOG
    off = plsc.cumsum(pe) - pe
    start_v[...] = off + prior
    off_v[...] = off
    end_v[...] = off + pe

    for i in range(TPS // NLANE):
        v = sid_v[pl.ds(i * NLANE, NLANE)]
        g = plsc.load_gather(start_v, [v])
        dst_v[pl.ds(i * NLANE, NLANE)] = g + rank_v[pl.ds(i * NLANE, NLANE)]

    used = plsc.load_gather(end_v, [jnp.full((NLANE,), E - 1, jnp.int32)])
    used = used >> BPLOG
    for cb in (0, 16):
        kvec = lax.iota(jnp.int32, NLANE) + cb
        sp = kvec * BP
        gid = jnp.zeros((NLANE,), jnp.int32)
        for e in range(1, E):
            be = plsc.load_gather(off_v, [jnp.full((NLANE,), e, jnp.int32)])
            gid = gid + (sp >= be).astype(jnp.int32)
        gid = jnp.where(kvec == NBLKP - 1, used, gid)
        blk_v[pl.ds(cb, NLANE)] = gid

    @pl.when(jnp.logical_and(c == 0, s == 0))
    def _():
        pltpu.sync_copy(blk_v, blk_hbm)

    def do_half(lo):
        pltpu.sync_copy(dst_v.at[pl.ds(lo, 128)],
                        dst_hbm.at[pl.ds(base + lo, 128)])
        bufs = (xbuf_a, xbuf_b)
        sems = (sem_a, sem_b)
        loads = [None, None]
        loads[0] = pltpu.async_copy(
            x_hbm.at[pl.ds(base + lo, NLANE)], bufs[0], sems[0])
        for j in range(8):
            p = j % 2
            if j < 7:
                loads[1 - p] = pltpu.async_copy(
                    x_hbm.at[pl.ds(base + lo + (j + 1) * NLANE, NLANE)],
                    bufs[1 - p], sems[1 - p])
            loads[p].wait()
            idx = dst_v[pl.ds(lo + j * NLANE, NLANE)]
            pltpu.async_copy(bufs[p], xs_hbm.at[idx], sems[p]).wait()

    @pl.when(c == 0)
    def _():
        do_half(0)

    @pl.when(c == 1)
    def _():
        do_half(128)


def _route_and_scatter(sid, x):
    f = pl.kernel(
        _route_body,
        compiler_params=pltpu.CompilerParams(needs_layout_passes=False),
        out_type=(
            jax.ShapeDtypeStruct((NP, D), jnp.float32),
            jax.ShapeDtypeStruct((N,), jnp.int32),
            jax.ShapeDtypeStruct((NBLKP,), jnp.int32),
        ),
        mesh=_sc_mesh(),
        scratch_types=[
            pltpu.VMEM((TPS,), jnp.int32),
            pltpu.VMEM((TPS,), jnp.int32),
            pltpu.VMEM((TPS,), jnp.int32),
            pltpu.VMEM((NLANE,), jnp.int32),
            pltpu.VMEM((256,), jnp.int32),
            pltpu.VMEM((NLANE,), jnp.int32),
            pltpu.VMEM((NLANE,), jnp.int32),
            pltpu.VMEM((NLANE,), jnp.int32),
            pltpu.VMEM((NBLKP,), jnp.int32),
            pltpu.VMEM((NLANE, D), jnp.float32),
            pltpu.VMEM((NLANE, D), jnp.float32),
            pltpu.VMEM_SHARED((256,), jnp.int32),
            pltpu.SemaphoreType.DMA,
            pltpu.SemaphoreType.DMA,
        ],
    )
    return f(sid, x)


def _mm_body(blk_ref, x_ref, w_ref, b_ref, y_ref):
    @pl.when(pl.program_id(0) < blk_ref[NBLKP - 1])
    def _():
        logits = jax.lax.dot_general(
            x_ref[...].astype(jnp.bfloat16), w_ref[0].astype(jnp.bfloat16),
            (((1,), (1,)), ((), ())),
            preferred_element_type=jnp.float32,
        ) + b_ref[0]
        y_ref[...] = jnp.concatenate(
            [logits, jnp.zeros((BP, CP - C), jnp.float32)], axis=1)


def _grouped_matmul(blk, xs, W, b3):
    grid_spec = pltpu.PrefetchScalarGridSpec(
        num_scalar_prefetch=1,
        grid=(NBLK,),
        in_specs=[
            pl.BlockSpec((BP, D), lambda k, g: (k, 0)),
            pl.BlockSpec((1, C, D), lambda k, g: (g[k], 0, 0)),
            pl.BlockSpec((1, 1, C), lambda k, g: (g[k], 0, 0)),
        ],
        out_specs=pl.BlockSpec((BP, CP), lambda k, g: (k, 0)),
    )
    return pl.pallas_call(
        _mm_body,
        grid_spec=grid_spec,
        out_shape=jax.ShapeDtypeStruct((NP, CP), jnp.float32),
        compiler_params=pltpu.CompilerParams(
            dimension_semantics=("arbitrary",),
        ),
    )(blk, xs, W, b3)


def _gather_body(y_hbm, dst_hbm, out_hbm, dst_v, rows_a, rows_b,
                 sem_a, sem_b):
    c = lax.axis_index("c")
    s = lax.axis_index("s")
    tok0 = s * TPS + c * 128
    pltpu.sync_copy(dst_hbm.at[pl.ds(tok0, 128)], dst_v)
    bufs = (rows_a, rows_b)
    sems = (sem_a, sem_b)
    loads = [None, None]
    idx0 = dst_v[pl.ds(0, NLANE)]
    loads[0] = pltpu.async_copy(y_hbm.at[idx0], bufs[0], sems[0])
    for j in range(8):
        p = j % 2
        if j < 7:
            idx = dst_v[pl.ds((j + 1) * NLANE, NLANE)]
            loads[1 - p] = pltpu.async_copy(y_hbm.at[idx], bufs[1 - p], sems[1 - p])
        loads[p].wait()
        pltpu.sync_copy(bufs[p], out_hbm.at[pl.ds(tok0 + j * NLANE, NLANE)])


def _gather_back(y, dst):
    f = pl.kernel(
        _gather_body,
        compiler_params=pltpu.CompilerParams(needs_layout_passes=False),
        out_type=jax.ShapeDtypeStruct((N, CP), jnp.float32),
        mesh=_sc_mesh(),
        scratch_types=[
            pltpu.VMEM((128,), jnp.int32),
            pltpu.VMEM((NLANE, CP), jnp.float32),
            pltpu.VMEM((NLANE, CP), jnp.float32),
            pltpu.SemaphoreType.DMA,
            pltpu.SemaphoreType.DMA,
        ],
    )
    return f(y, dst)


def _trim_body(full_ref, out_ref):
    out_ref[...] = full_ref[:, :C]


def _trim(full):
    bn = 512
    return pl.pallas_call(
        _trim_body,
        grid=(N // bn,),
        in_specs=[pl.BlockSpec((bn, CP), lambda i: (i, 0))],
        out_specs=pl.BlockSpec((bn, C), lambda i: (i, 0)),
        out_shape=jax.ShapeDtypeStruct((N, C), jnp.float32),
    )(full)


def kernel(x, system_id, W, b):
    sid = system_id.astype(jnp.int32)
    b3 = b.reshape(E, 1, C)
    xs, dst, blk = _route_and_scatter(sid, x)
    y = _grouped_matmul(blk, xs, W, b3)
    return _trim(_gather_back(y, dst))


# speedup vs baseline: 1.2271x; 1.0885x over previous
import functools

import jax
import jax.numpy as jnp
from jax import lax
from jax.experimental import pallas as pl
from jax.experimental.pallas import tpu as pltpu
from jax.experimental.pallas import tpu_sc as plsc

N = 4096
D = 1024
C = 1000
E = 8

BP = 256
BPLOG = 8
NP = N + E * BP
NBLK = NP // BP
NBLKP = 32
CP = 1024
CH = CP // 2
TPS = 256
NLANE = 16


def _sc_mesh():
    return plsc.VectorSubcoreMesh(core_axis_name="c", subcore_axis_name="s")


def _route_body(sid_hbm, x_hbm, xs_hbm, dst_hbm, blk_hbm,
                sid_v, rank_v, dst_v, cnt_v, allc_v, start_v, off_v, end_v,
                blk_v, xbuf_a, xbuf_b, sh_cnt, sem_a, sem_b):
    c = lax.axis_index("c")
    s = lax.axis_index("s")
    base = s * TPS
    pltpu.sync_copy(sid_hbm.at[pl.ds(base, TPS)], sid_v)

    lane = lax.iota(jnp.int32, NLANE)
    counts = [jnp.zeros((NLANE,), jnp.int32) for _ in range(E)]
    for i in range(TPS // NLANE):
        v = sid_v[pl.ds(i * NLANE, NLANE)]
        rank = jnp.zeros((NLANE,), jnp.int32)
        for e in range(E):
            m = v == e
            cs = plsc.cumsum(m.astype(jnp.int32))
            rank = jnp.where(m, counts[e] + cs - 1, rank)
            counts[e] = counts[e] + plsc.all_reduce_population_count(m)
        rank_v[pl.ds(i * NLANE, NLANE)] = rank

    cvec = jnp.zeros((NLANE,), jnp.int32)
    for e in range(E):
        cvec = jnp.where(lane == e, counts[e], cvec)
    cnt_v[...] = cvec
    pltpu.sync_copy(cnt_v, sh_cnt.at[pl.ds(s * NLANE, NLANE)])
    plsc.subcore_barrier()
    pltpu.sync_copy(sh_cnt, allc_v)

    total = jnp.zeros((NLANE,), jnp.int32)
    prior = jnp.zeros((NLANE,), jnp.int32)
    for t in range(16):
        row = allc_v[pl.ds(t * NLANE, NLANE)]
        total = total + row
        tm = jnp.full((NLANE,), t, jnp.int32) < s
        prior = prior + jnp.where(tm, row, jnp.zeros((NLANE,), jnp.int32))

    pe = ((total + (BP - 1)) >> BPLOG) << BPLOG
    off = plsc.cumsum(pe) - pe
    start_v[...] = off + prior
    off_v[...] = off
    end_v[...] = off + pe

    for i in range(TPS // NLANE):
        v = sid_v[pl.ds(i * NLANE, NLANE)]
        g = plsc.load_gather(start_v, [v])
        dst_v[pl.ds(i * NLANE, NLANE)] = g + rank_v[pl.ds(i * NLANE, NLANE)]

    used = plsc.load_gather(end_v, [jnp.full((NLANE,), E - 1, jnp.int32)])
    used = used >> BPLOG
    for cb in (0, 16):
        kvec = lax.iota(jnp.int32, NLANE) + cb
        sp = kvec * BP
        gid = jnp.zeros((NLANE,), jnp.int32)
        for e in range(1, E):
            be = plsc.load_gather(off_v, [jnp.full((NLANE,), e, jnp.int32)])
            gid = gid + (sp >= be).astype(jnp.int32)
        gid = jnp.where(kvec == NBLKP - 1, used, gid)
        blk_v[pl.ds(cb, NLANE)] = gid

    @pl.when(jnp.logical_and(c == 0, s == 0))
    def _():
        pltpu.sync_copy(blk_v, blk_hbm)

    def do_half(lo):
        pltpu.sync_copy(dst_v.at[pl.ds(lo, 128)],
                        dst_hbm.at[pl.ds(base + lo, 128)])
        bufs = (xbuf_a, xbuf_b)
        sems = (sem_a, sem_b)
        loads = [None, None]
        loads[0] = pltpu.async_copy(
            x_hbm.at[pl.ds(base + lo, NLANE)], bufs[0], sems[0])
        for j in range(8):
            p = j % 2
            if j < 7:
                loads[1 - p] = pltpu.async_copy(
                    x_hbm.at[pl.ds(base + lo + (j + 1) * NLANE, NLANE)],
                    bufs[1 - p], sems[1 - p])
            loads[p].wait()
            idx = dst_v[pl.ds(lo + j * NLANE, NLANE)]
            pltpu.async_copy(bufs[p], xs_hbm.at[idx], sems[p]).wait()

    @pl.when(c == 0)
    def _():
        do_half(0)

    @pl.when(c == 1)
    def _():
        do_half(128)


def _route_and_scatter(sid, x):
    f = pl.kernel(
        _route_body,
        compiler_params=pltpu.CompilerParams(needs_layout_passes=False),
        out_type=(
            jax.ShapeDtypeStruct((NP, D), jnp.float32),
            jax.ShapeDtypeStruct((N,), jnp.int32),
            jax.ShapeDtypeStruct((NBLKP,), jnp.int32),
        ),
        mesh=_sc_mesh(),
        scratch_types=[
            pltpu.VMEM((TPS,), jnp.int32),
            pltpu.VMEM((TPS,), jnp.int32),
            pltpu.VMEM((TPS,), jnp.int32),
            pltpu.VMEM((NLANE,), jnp.int32),
            pltpu.VMEM((256,), jnp.int32),
            pltpu.VMEM((NLANE,), jnp.int32),
            pltpu.VMEM((NLANE,), jnp.int32),
            pltpu.VMEM((NLANE,), jnp.int32),
            pltpu.VMEM((NBLKP,), jnp.int32),
            pltpu.VMEM((NLANE, D), jnp.float32),
            pltpu.VMEM((NLANE, D), jnp.float32),
            pltpu.VMEM_SHARED((256,), jnp.int32),
            pltpu.SemaphoreType.DMA,
            pltpu.SemaphoreType.DMA,
        ],
    )
    return f(sid, x)


def _mm_body(blk_ref, x_ref, w_ref, b_ref, y_ref):
    @pl.when(pl.program_id(0) < blk_ref[NBLKP - 1])
    def _():
        logits = jax.lax.dot_general(
            x_ref[...].astype(jnp.bfloat16), w_ref[0].astype(jnp.bfloat16),
            (((1,), (1,)), ((), ())),
            preferred_element_type=jnp.float32,
        ) + b_ref[0]
        full = jnp.concatenate(
            [logits, jnp.zeros((BP, CP - C), jnp.float32)], axis=1
        ).astype(jnp.bfloat16)
        lo = jax.lax.bitcast_convert_type(
            full[:, :CH], jnp.uint16).astype(jnp.uint32)
        hi = jax.lax.bitcast_convert_type(
            full[:, CH:], jnp.uint16).astype(jnp.uint32)
        y_ref[...] = lo | (hi << 16)


def _grouped_matmul(blk, xs, W, b3):
    grid_spec = pltpu.PrefetchScalarGridSpec(
        num_scalar_prefetch=1,
        grid=(NBLK,),
        in_specs=[
            pl.BlockSpec((BP, D), lambda k, g: (k, 0)),
            pl.BlockSpec((1, C, D), lambda k, g: (g[k], 0, 0)),
            pl.BlockSpec((1, 1, C), lambda k, g: (g[k], 0, 0)),
        ],
        out_specs=pl.BlockSpec((BP, CH), lambda k, g: (k, 0)),
    )
    return pl.pallas_call(
        _mm_body,
        grid_spec=grid_spec,
        out_shape=jax.ShapeDtypeStruct((NP, CH), jnp.uint32),
        compiler_params=pltpu.CompilerParams(
            dimension_semantics=("arbitrary",),
        ),
    )(blk, xs, W, b3)


def _gather_body(y_hbm, dst_hbm, out_hbm, dst_v, rows_a, rows_b,
                 sem_a, sem_b):
    c = lax.axis_index("c")
    s = lax.axis_index("s")
    tok0 = s * TPS + c * 128
    pltpu.sync_copy(dst_hbm.at[pl.ds(tok0, 128)], dst_v)
    bufs = (rows_a, rows_b)
    sems = (sem_a, sem_b)
    loads = [None, None]
    idx0 = dst_v[pl.ds(0, NLANE)]
    loads[0] = pltpu.async_copy(y_hbm.at[idx0], bufs[0], sems[0])
    for j in range(8):
        p = j % 2
        if j < 7:
            idx = dst_v[pl.ds((j + 1) * NLANE, NLANE)]
            loads[1 - p] = pltpu.async_copy(y_hbm.at[idx], bufs[1 - p], sems[1 - p])
        loads[p].wait()
        pltpu.sync_copy(bufs[p], out_hbm.at[pl.ds(tok0 + j * NLANE, NLANE)])


def _gather_back(y, dst):
    f = pl.kernel(
        _gather_body,
        compiler_params=pltpu.CompilerParams(needs_layout_passes=False),
        out_type=jax.ShapeDtypeStruct((N, CH), jnp.uint32),
        mesh=_sc_mesh(),
        scratch_types=[
            pltpu.VMEM((128,), jnp.int32),
            pltpu.VMEM((NLANE, CH), jnp.uint32),
            pltpu.VMEM((NLANE, CH), jnp.uint32),
            pltpu.SemaphoreType.DMA,
            pltpu.SemaphoreType.DMA,
        ],
    )
    return f(y, dst)


def _trim_body(packed_ref, out_ref):
    u = packed_ref[...]
    lo = jax.lax.bitcast_convert_type(
        (u & jnp.uint32(0xFFFF)).astype(jnp.uint16), jnp.bfloat16)
    hi = jax.lax.bitcast_convert_type(
        (u >> 16).astype(jnp.uint16), jnp.bfloat16)
    full = jnp.concatenate([lo, hi], axis=1).astype(jnp.float32)
    out_ref[...] = full[:, :C]


def _trim(packed):
    bn = 512
    return pl.pallas_call(
        _trim_body,
        grid=(N // bn,),
        in_specs=[pl.BlockSpec((bn, CH), lambda i: (i, 0))],
        out_specs=pl.BlockSpec((bn, C), lambda i: (i, 0)),
        out_shape=jax.ShapeDtypeStruct((N, C), jnp.float32),
    )(packed)


def kernel(x, system_id, W, b):
    sid = system_id.astype(jnp.int32)
    b3 = b.reshape(E, 1, C)
    xs, dst, blk = _route_and_scatter(sid, x)
    y = _grouped_matmul(blk, xs, W, b3)
    return _trim(_gather_back(y, dst))


# speedup vs baseline: 1.2542x; 1.0221x over previous
import functools

import jax
import jax.numpy as jnp
from jax import lax
from jax.experimental import pallas as pl
from jax.experimental.pallas import tpu as pltpu
from jax.experimental.pallas import tpu_sc as plsc

N = 4096
D = 1024
C = 1000
E = 8

BP = 256
BPLOG = 8
NP = N + E * BP
NBLK = NP // BP
NBLKP = 32
CP = 1024
CH = CP // 2
TPS = 256
NLANE = 16


def _sc_mesh():
    return plsc.VectorSubcoreMesh(core_axis_name="c", subcore_axis_name="s")


def _route_body(sid_hbm, x_hbm, xs_hbm, dst_hbm, blk_hbm,
                sid_v, rank_v, dst_v, cnt_v, allc_v, start_v, off_v, end_v,
                blk_v, xbuf_a, xbuf_b, sh_cnt, sem_a, sem_b):
    c = lax.axis_index("c")
    s = lax.axis_index("s")
    base = s * TPS
    pltpu.sync_copy(sid_hbm.at[pl.ds(base, TPS)], sid_v)

    lane = lax.iota(jnp.int32, NLANE)
    counts = [jnp.zeros((NLANE,), jnp.int32) for _ in range(E)]
    for i in range(TPS // NLANE):
        v = sid_v[pl.ds(i * NLANE, NLANE)]
        rank = jnp.zeros((NLANE,), jnp.int32)
        for e in range(E):
            m = v == e
            cs = plsc.cumsum(m.astype(jnp.int32))
            rank = jnp.where(m, counts[e] + cs - 1, rank)
            counts[e] = counts[e] + plsc.all_reduce_population_count(m)
        rank_v[pl.ds(i * NLANE, NLANE)] = rank

    cvec = jnp.zeros((NLANE,), jnp.int32)
    for e in range(E):
        cvec = jnp.where(lane == e, counts[e], cvec)
    cnt_v[...] = cvec
    pltpu.sync_copy(cnt_v, sh_cnt.at[pl.ds(s * NLANE, NLANE)])
    plsc.subcore_barrier()
    pltpu.sync_copy(sh_cnt, allc_v)

    total = jnp.zeros((NLANE,), jnp.int32)
    prior = jnp.zeros((NLANE,), jnp.int32)
    for t in range(16):
        row = allc_v[pl.ds(t * NLANE, NLANE)]
        total = total + row
        tm = jnp.full((NLANE,), t, jnp.int32) < s
        prior = prior + jnp.where(tm, row, jnp.zeros((NLANE,), jnp.int32))

    pe = ((total + (BP - 1)) >> BPLOG) << BPLOG
    off = plsc.cumsum(pe) - pe
    start_v[...] = off + prior
    off_v[...] = off
    end_v[...] = off + pe

    for i in range(TPS // NLANE):
        v = sid_v[pl.ds(i * NLANE, NLANE)]
        g = plsc.load_gather(start_v, [v])
        dst_v[pl.ds(i * NLANE, NLANE)] = g + rank_v[pl.ds(i * NLANE, NLANE)]

    used = plsc.load_gather(end_v, [jnp.full((NLANE,), E - 1, jnp.int32)])
    used = used >> BPLOG
    for cb in (0, 16):
        kvec = lax.iota(jnp.int32, NLANE) + cb
        sp = kvec * BP
        gid = jnp.zeros((NLANE,), jnp.int32)
        for e in range(1, E):
            be = plsc.load_gather(off_v, [jnp.full((NLANE,), e, jnp.int32)])
            gid = gid + (sp >= be).astype(jnp.int32)
        gid = jnp.where(kvec == NBLKP - 1, used, gid)
        blk_v[pl.ds(cb, NLANE)] = gid

    @pl.when(jnp.logical_and(c == 0, s == 0))
    def _():
        pltpu.sync_copy(blk_v, blk_hbm)

    def do_half(lo):
        pltpu.sync_copy(dst_v.at[pl.ds(lo, 128)],
                        dst_hbm.at[pl.ds(base + lo, 128)])
        bufs = (xbuf_a, xbuf_b)
        sems = (sem_a, sem_b)
        loads = [None, None]
        loads[0] = pltpu.async_copy(
            x_hbm.at[pl.ds(base + lo, NLANE)], bufs[0], sems[0])
        for j in range(8):
            p = j % 2
            if j < 7:
                loads[1 - p] = pltpu.async_copy(
                    x_hbm.at[pl.ds(base + lo + (j + 1) * NLANE, NLANE)],
                    bufs[1 - p], sems[1 - p])
            loads[p].wait()
            idx = dst_v[pl.ds(lo + j * NLANE, NLANE)]
            pltpu.async_copy(bufs[p], xs_hbm.at[idx], sems[p]).wait()

    @pl.when(c == 0)
    def _():
        do_half(0)

    @pl.when(c == 1)
    def _():
        do_half(128)


def _route_and_scatter(sid, x):
    f = pl.kernel(
        _route_body,
        compiler_params=pltpu.CompilerParams(needs_layout_passes=False),
        out_type=(
            jax.ShapeDtypeStruct((NP, D), jnp.float32),
            jax.ShapeDtypeStruct((N,), jnp.int32),
            jax.ShapeDtypeStruct((NBLKP,), jnp.int32),
        ),
        mesh=_sc_mesh(),
        scratch_types=[
            pltpu.VMEM((TPS,), jnp.int32),
            pltpu.VMEM((TPS,), jnp.int32),
            pltpu.VMEM((TPS,), jnp.int32),
            pltpu.VMEM((NLANE,), jnp.int32),
            pltpu.VMEM((256,), jnp.int32),
            pltpu.VMEM((NLANE,), jnp.int32),
            pltpu.VMEM((NLANE,), jnp.int32),
            pltpu.VMEM((NLANE,), jnp.int32),
            pltpu.VMEM((NBLKP,), jnp.int32),
            pltpu.VMEM((NLANE, D), jnp.float32),
            pltpu.VMEM((NLANE, D), jnp.float32),
            pltpu.VMEM_SHARED((256,), jnp.int32),
            pltpu.SemaphoreType.DMA,
            pltpu.SemaphoreType.DMA,
        ],
    )
    return f(sid, x)


def _mm_body(blk_ref, x_ref, w_ref, b_ref, y_ref):
    @pl.when(pl.program_id(0) < blk_ref[NBLKP - 1])
    def _():
        logits = jax.lax.dot_general(
            x_ref[...].astype(jnp.bfloat16), w_ref[0].astype(jnp.bfloat16),
            (((1,), (1,)), ((), ())),
            preferred_element_type=jnp.float32,
        ) + b_ref[0]
        full = jnp.concatenate(
            [logits, jnp.zeros((BP, CP - C), jnp.float32)], axis=1
        ).astype(jnp.bfloat16)
        lo = jax.lax.bitcast_convert_type(
            full[:, :CH], jnp.uint16).astype(jnp.uint32)
        hi = jax.lax.bitcast_convert_type(
            full[:, CH:], jnp.uint16).astype(jnp.uint32)
        y_ref[...] = lo | (hi << 16)


def _grouped_matmul(blk, xs, W, b3):
    grid_spec = pltpu.PrefetchScalarGridSpec(
        num_scalar_prefetch=1,
        grid=(NBLK,),
        in_specs=[
            pl.BlockSpec((BP, D), lambda k, g: (k, 0)),
            pl.BlockSpec((1, C, D), lambda k, g: (g[k], 0, 0)),
            pl.BlockSpec((1, 1, C), lambda k, g: (g[k], 0, 0)),
        ],
        out_specs=pl.BlockSpec((BP, CH), lambda k, g: (k, 0)),
    )
    return pl.pallas_call(
        _mm_body,
        grid_spec=grid_spec,
        out_shape=jax.ShapeDtypeStruct((NP, CH), jnp.uint32),
        compiler_params=pltpu.CompilerParams(
            dimension_semantics=("arbitrary",),
        ),
    )(blk, xs, W, b3)


def _gather_body(y_hbm, dst_hbm, out_hbm, dst_v, rows_a, rows_b,
                 sem_a, sem_b):
    c = lax.axis_index("c")
    s = lax.axis_index("s")
    tok0 = s * TPS + c * 128
    pltpu.sync_copy(dst_hbm.at[pl.ds(tok0, 128)], dst_v)
    bufs = (rows_a, rows_b)
    sems = (sem_a, sem_b)
    loads = [None, None]
    idx0 = dst_v[pl.ds(0, NLANE)]
    loads[0] = pltpu.async_copy(y_hbm.at[idx0], bufs[0], sems[0])
    for j in range(8):
        p = j % 2
        if j < 7:
            idx = dst_v[pl.ds((j + 1) * NLANE, NLANE)]
            loads[1 - p] = pltpu.async_copy(y_hbm.at[idx], bufs[1 - p], sems[1 - p])
        loads[p].wait()
        pltpu.sync_copy(bufs[p], out_hbm.at[pl.ds(tok0 + j * NLANE, NLANE)])


def _gather_back(y, dst):
    f = pl.kernel(
        _gather_body,
        compiler_params=pltpu.CompilerParams(needs_layout_passes=False),
        out_type=jax.ShapeDtypeStruct((N, CH), jnp.uint32),
        mesh=_sc_mesh(),
        scratch_types=[
            pltpu.VMEM((128,), jnp.int32),
            pltpu.VMEM((NLANE, CH), jnp.uint32),
            pltpu.VMEM((NLANE, CH), jnp.uint32),
            pltpu.SemaphoreType.DMA,
            pltpu.SemaphoreType.DMA,
        ],
    )
    return f(y, dst)


def _trim_body(packed_ref, out_ref):
    u = packed_ref[...]
    lo = jax.lax.bitcast_convert_type(
        (u & jnp.uint32(0xFFFF)).astype(jnp.uint16), jnp.bfloat16)
    hi = jax.lax.bitcast_convert_type(
        (u >> 16).astype(jnp.uint16), jnp.bfloat16)
    full = jnp.concatenate([lo, hi], axis=1).astype(jnp.float32)
    out_ref[...] = full[:, :C]


def _trim(packed):
    bn = 2048
    return pl.pallas_call(
        _trim_body,
        grid=(N // bn,),
        in_specs=[pl.BlockSpec((bn, CH), lambda i: (i, 0))],
        out_specs=pl.BlockSpec((bn, C), lambda i: (i, 0)),
        out_shape=jax.ShapeDtypeStruct((N, C), jnp.float32),
    )(packed)


def kernel(x, system_id, W, b):
    sid = system_id.astype(jnp.int32)
    b3 = b.reshape(E, 1, C)
    xs, dst, blk = _route_and_scatter(sid, x)
    y = _grouped_matmul(blk, xs, W, b3)
    return _trim(_gather_back(y, dst))
